# Initial kernel scaffold; baseline (speedup 1.0000x reference)
import jax
import jax.numpy as jnp
from jax.experimental import pallas as pl


def kernel(user_ids, item_ids, user_table, item_table):
    raise NotImplementedError("write your pallas kernel here")


# speedup vs baseline: 1.2716x; 1.2716x over previous
import functools

import jax
import jax.numpy as jnp
from jax import lax
from jax.experimental import pallas as pl
from jax.experimental.pallas import tpu as pltpu
from jax.experimental.pallas import tpu_sc as plsc

BATCH = 16384
DIM = 128
CHUNK = 128


@functools.cache
def _build():
    info = plsc.get_sparse_core_info()
    nc, ns = info.num_cores, info.num_subcores
    nw = nc * ns
    b_per_w = BATCH // nw
    ch = b_per_w // CHUNK

    mesh = plsc.VectorSubcoreMesh(core_axis_name="c", subcore_axis_name="s")

    @functools.partial(
        pl.kernel,
        mesh=mesh,
        out_type=(
            jax.ShapeDtypeStruct((BATCH, DIM), jnp.float32),
            jax.ShapeDtypeStruct((BATCH, DIM), jnp.float32),
        ),
        scratch_types=[
            pltpu.VMEM((ch, CHUNK), jnp.int32),
            pltpu.VMEM((ch, CHUNK), jnp.int32),
            pltpu.VMEM((2, CHUNK, DIM), jnp.float32),
            pltpu.SemaphoreType.DMA,
            pltpu.SemaphoreType.DMA,
            pltpu.SemaphoreType.DMA,
            pltpu.SemaphoreType.DMA,
        ],
    )
    def emb(uids_hbm, iids_hbm, utab_hbm, itab_hbm, uout_hbm, iout_hbm,
            uidx_v, iidx_v, rows_v, gsem0, gsem1, wsem0, wsem1):
        wid = lax.axis_index("s") * nc + lax.axis_index("c")
        rbase = wid * ch
        obase = wid * b_per_w

        pltpu.sync_copy(uids_hbm.at[pl.ds(rbase, ch)], uidx_v)
        pltpu.sync_copy(iids_hbm.at[pl.ds(rbase, ch)], iidx_v)

        gsems = (gsem0, gsem1)
        wsems = (wsem0, wsem1)
        tasks = [(utab_hbm, uidx_v, uout_hbm, j) for j in range(ch)] + \
                [(itab_hbm, iidx_v, iout_hbm, j) for j in range(ch)]
        n = len(tasks)

        gh = {}
        wb = {}

        def start_gather(t):
            tab, idxv, _, j = tasks[t]
            b = t % 2
            gh[t] = pltpu.async_copy(tab.at[idxv.at[j]], rows_v.at[b], gsems[b])

        start_gather(0)
        for t in range(n):
            _, _, outr, j = tasks[t]
            b = t % 2
            gh[t].wait()
            if t + 1 < n:
                if t - 1 >= 0:
                    wb[t - 1].wait()
                start_gather(t + 1)
            dst = outr.at[pl.ds(obase + j * CHUNK, CHUNK)]
            wb[t] = pltpu.async_copy(rows_v.at[b], dst, wsems[b])
        wb[n - 2].wait()
        wb[n - 1].wait()

    return emb


def kernel(user_ids, item_ids, user_table, item_table):
    uids = user_ids.astype(jnp.int32).reshape(BATCH // CHUNK, CHUNK)
    iids = item_ids.astype(jnp.int32).reshape(BATCH // CHUNK, CHUNK)
    u_out, i_out = _build()(uids, iids, user_table, item_table)
    return (u_out, u_out, i_out)


# speedup vs baseline: 1.3798x; 1.0851x over previous
import functools

import jax
import jax.numpy as jnp
from jax import lax
from jax.experimental import pallas as pl
from jax.experimental.pallas import tpu as pltpu
from jax.experimental.pallas import tpu_sc as plsc

BATCH = 16384
DIM = 128
CHUNK = 128
NBUF = 4


@functools.cache
def _build():
    info = plsc.get_sparse_core_info()
    nc, ns = info.num_cores, info.num_subcores
    nw = nc * ns
    b_per_w = BATCH // nw
    ch = b_per_w // CHUNK

    mesh = plsc.VectorSubcoreMesh(core_axis_name="c", subcore_axis_name="s")

    @functools.partial(
        pl.kernel,
        mesh=mesh,
        out_type=(
            jax.ShapeDtypeStruct((BATCH, DIM), jnp.float32),
            jax.ShapeDtypeStruct((BATCH, DIM), jnp.float32),
        ),
        scratch_types=[
            pltpu.VMEM((ch, CHUNK), jnp.int32),
            pltpu.VMEM((ch, CHUNK), jnp.int32),
            pltpu.VMEM((NBUF, CHUNK, DIM), jnp.float32),
            pltpu.SemaphoreType.DMA,
            pltpu.SemaphoreType.DMA,
            pltpu.SemaphoreType.DMA,
            pltpu.SemaphoreType.DMA,
            pltpu.SemaphoreType.DMA,
            pltpu.SemaphoreType.DMA,
            pltpu.SemaphoreType.DMA,
            pltpu.SemaphoreType.DMA,
        ],
    )
    def emb(uids_hbm, iids_hbm, utab_hbm, itab_hbm, uout_hbm, iout_hbm,
            uidx_v, iidx_v, rows_v, g0, g1, g2, g3, w0, w1, w2, w3):
        wid = lax.axis_index("s") * nc + lax.axis_index("c")
        rbase = wid * ch
        obase = wid * b_per_w

        pltpu.sync_copy(uids_hbm.at[pl.ds(rbase, ch)], uidx_v)
        pltpu.sync_copy(iids_hbm.at[pl.ds(rbase, ch)], iidx_v)

        gsems = (g0, g1, g2, g3)
        wsems = (w0, w1, w2, w3)
        tasks = [(utab_hbm, uidx_v, uout_hbm, j) for j in range(ch)] + \
                [(itab_hbm, iidx_v, iout_hbm, j) for j in range(ch)]
        n = len(tasks)

        gh = {}
        wb = {}

        def start_gather(t):
            tab, idxv, _, j = tasks[t]
            b = t % NBUF
            gh[t] = pltpu.async_copy(tab.at[idxv.at[j]], rows_v.at[b], gsems[b])

        for t in range(min(NBUF, n)):
            start_gather(t)
        for t in range(n):
            _, _, outr, j = tasks[t]
            b = t % NBUF
            gh[t].wait()
            dst = outr.at[pl.ds(obase + j * CHUNK, CHUNK)]
            wb[t] = pltpu.async_copy(rows_v.at[b], dst, wsems[b])
            if t + NBUF < n:
                wb[t].wait()
                start_gather(t + NBUF)
        for t in range(max(0, n - NBUF), n):
            wb[t].wait()

    return emb


def kernel(user_ids, item_ids, user_table, item_table):
    uids = user_ids.astype(jnp.int32).reshape(BATCH // CHUNK, CHUNK)
    iids = item_ids.astype(jnp.int32).reshape(BATCH // CHUNK, CHUNK)
    u_out, i_out = _build()(uids, iids, user_table, item_table)
    return (u_out, u_out, i_out)


# speedup vs baseline: 1.4954x; 1.0838x over previous
import functools

import jax
import jax.numpy as jnp
from jax import lax
from jax.experimental import pallas as pl
from jax.experimental.pallas import tpu as pltpu
from jax.experimental.pallas import tpu_sc as plsc

BATCH = 16384
DIM = 128
CHUNK = 128
NBUF = 4


@functools.cache
def _build():
    info = plsc.get_sparse_core_info()
    nc, ns = info.num_cores, info.num_subcores
    nw = nc * ns
    b_per_w = BATCH // nw
    ch = b_per_w // CHUNK

    mesh = plsc.VectorSubcoreMesh(core_axis_name="c", subcore_axis_name="s")

    @functools.partial(
        pl.kernel,
        mesh=mesh,
        out_type=(
            jax.ShapeDtypeStruct((BATCH, DIM), jnp.float32),
            jax.ShapeDtypeStruct((BATCH, DIM), jnp.float32),
            jax.ShapeDtypeStruct((BATCH, DIM), jnp.float32),
        ),
        scratch_types=[
            pltpu.VMEM((ch, CHUNK), jnp.int32),
            pltpu.VMEM((ch, CHUNK), jnp.int32),
            pltpu.VMEM((NBUF, CHUNK, DIM), jnp.float32),
            pltpu.SemaphoreType.DMA,
            pltpu.SemaphoreType.DMA,
            pltpu.SemaphoreType.DMA,
            pltpu.SemaphoreType.DMA,
            pltpu.SemaphoreType.DMA,
            pltpu.SemaphoreType.DMA,
            pltpu.SemaphoreType.DMA,
            pltpu.SemaphoreType.DMA,
            pltpu.SemaphoreType.DMA,
        ],
    )
    def emb(uids_hbm, iids_hbm, utab_hbm, itab_hbm,
            uout_a_hbm, uout_b_hbm, iout_hbm,
            uidx_v, iidx_v, rows_v, isem, g0, g1, g2, g3, w0, w1, w2, w3):
        wid = lax.axis_index("s") * nc + lax.axis_index("c")
        rbase = wid * ch
        obase = wid * b_per_w

        ih_u = pltpu.async_copy(uids_hbm.at[pl.ds(rbase, ch)], uidx_v, isem)
        ih_i = pltpu.async_copy(iids_hbm.at[pl.ds(rbase, ch)], iidx_v, isem)
        ih_u.wait()
        ih_i.wait()

        gsems = (g0, g1, g2, g3)
        wsems = (w0, w1, w2, w3)
        tasks = [(utab_hbm, uidx_v, (uout_a_hbm, uout_b_hbm), j)
                 for j in range(ch)] + \
                [(itab_hbm, iidx_v, (iout_hbm,), j) for j in range(ch)]
        n = len(tasks)

        gh = {}
        wb = {}

        def start_gather(t):
            tab, idxv, _, j = tasks[t]
            b = t % NBUF
            gh[t] = pltpu.async_copy(tab.at[idxv.at[j]], rows_v.at[b], gsems[b])

        for t in range(min(NBUF, n)):
            start_gather(t)
        for t in range(n):
            _, _, outs, j = tasks[t]
            b = t % NBUF
            gh[t].wait()
            sl = pl.ds(obase + j * CHUNK, CHUNK)
            wb[t] = [pltpu.async_copy(rows_v.at[b], outr.at[sl], wsems[b])
                     for outr in outs]
            if t + NBUF < n:
                for h in wb[t]:
                    h.wait()
                start_gather(t + NBUF)
        for t in range(max(0, n - NBUF), n):
            for h in wb[t]:
                h.wait()

    return emb


def kernel(user_ids, item_ids, user_table, item_table):
    uids = user_ids.astype(jnp.int32).reshape(BATCH // CHUNK, CHUNK)
    iids = item_ids.astype(jnp.int32).reshape(BATCH // CHUNK, CHUNK)
    return _build()(uids, iids, user_table, item_table)


# speedup vs baseline: 1.5735x; 1.0522x over previous
import functools

import jax
import jax.numpy as jnp
from jax import lax
from jax.experimental import pallas as pl
from jax.experimental.pallas import tpu as pltpu
from jax.experimental.pallas import tpu_sc as plsc

BATCH = 16384
DIM = 128
CHUNK = 128


@functools.cache
def _build():
    info = plsc.get_sparse_core_info()
    nc, ns = info.num_cores, info.num_subcores
    nw = nc * ns
    b_per_w = BATCH // nw
    ch = b_per_w // CHUNK

    mesh = plsc.VectorSubcoreMesh(core_axis_name="c", subcore_axis_name="s")

    @functools.partial(
        pl.kernel,
        mesh=mesh,
        out_type=(
            jax.ShapeDtypeStruct((BATCH, DIM), jnp.float32),
            jax.ShapeDtypeStruct((BATCH, DIM), jnp.float32),
            jax.ShapeDtypeStruct((BATCH, DIM), jnp.float32),
        ),
        scratch_types=[
            pltpu.VMEM((ch, CHUNK), jnp.int32),
            pltpu.VMEM((ch, CHUNK), jnp.int32),
            pltpu.VMEM((b_per_w, DIM), jnp.float32),
            pltpu.VMEM((2, CHUNK, DIM), jnp.float32),
            pltpu.SemaphoreType.DMA,
            pltpu.SemaphoreType.DMA,
            pltpu.SemaphoreType.DMA,
            pltpu.SemaphoreType.DMA,
            pltpu.SemaphoreType.DMA,
            pltpu.SemaphoreType.DMA,
            pltpu.SemaphoreType.DMA,
            pltpu.SemaphoreType.DMA,
        ],
    )
    def emb(uids_hbm, iids_hbm, utab_hbm, itab_hbm,
            uout_a_hbm, uout_b_hbm, iout_hbm,
            uidx_v, iidx_v, urows_v, irows_v,
            isem, gu, gi0, gi1, wua, wub, wi0, wi1):
        wid = lax.axis_index("s") * nc + lax.axis_index("c")
        rbase = wid * ch
        obase = wid * b_per_w

        ih_u = pltpu.async_copy(uids_hbm.at[pl.ds(rbase, ch)], uidx_v, isem)
        ih_i = pltpu.async_copy(iids_hbm.at[pl.ds(rbase, ch)], iidx_v, isem)
        ih_u.wait()
        ih_i.wait()

        gisems = (gi0, gi1)
        wisems = (wi0, wi1)

        ug = [pltpu.async_copy(utab_hbm.at[uidx_v.at[j]],
                               urows_v.at[pl.ds(j * CHUNK, CHUNK)], gu)
              for j in range(ch)]
        ig = {j: pltpu.async_copy(itab_hbm.at[iidx_v.at[j]],
                                  irows_v.at[j % 2], gisems[j % 2])
              for j in range(2)}

        iwb = {}

        def item_writeback(j):
            sl = pl.ds(obase + j * CHUNK, CHUNK)
            iwb[j] = pltpu.async_copy(irows_v.at[j % 2], iout_hbm.at[sl],
                                      wisems[j % 2])

        ig[0].wait()
        item_writeback(0)
        ig[1].wait()
        item_writeback(1)

        for h in ug:
            h.wait()
        osl = pl.ds(obase, b_per_w)
        uwa = pltpu.async_copy(urows_v, uout_a_hbm.at[osl], wua)
        uwb = pltpu.async_copy(urows_v, uout_b_hbm.at[osl], wub)

        for j in range(2, ch):
            iwb[j - 2].wait()
            ig[j] = pltpu.async_copy(itab_hbm.at[iidx_v.at[j]],
                                     irows_v.at[j % 2], gisems[j % 2])
        for j in range(2, ch):
            ig[j].wait()
            item_writeback(j)

        uwa.wait()
        uwb.wait()
        for j in range(ch - 2, ch):
            iwb[j].wait()

    return emb


def kernel(user_ids, item_ids, user_table, item_table):
    uids = user_ids.astype(jnp.int32).reshape(BATCH // CHUNK, CHUNK)
    iids = item_ids.astype(jnp.int32).reshape(BATCH // CHUNK, CHUNK)
    return _build()(uids, iids, user_table, item_table)
